# baseline (speedup 1.0000x reference)
import functools

import jax
import jax.numpy as jnp
from jax import lax
from jax.experimental import pallas as pl
from jax.experimental.pallas import tpu as pltpu
from jax.experimental.pallas import tpu_sc as plsc

VOCAB = 1000000
EMBED = 32
B, T = 4096, 200
N = B * T

_info = plsc.get_sparse_core_info()
NC, NS = _info.num_cores, _info.num_subcores
NW = NC * NS
BLK = 128
SUB = 10
GSIZE = BLK * SUB
PER_W = N // NW
NGROUP = PER_W // GSIZE

_mesh = plsc.VectorSubcoreMesh(core_axis_name="c", subcore_axis_name="s")


@functools.partial(
    pl.kernel,
    mesh=_mesh,
    out_type=jax.ShapeDtypeStruct((T, EMBED // 8, B // BLK, 8, BLK), jnp.float32),
    compiler_params=pltpu.CompilerParams(
        use_tc_tiling_on_sc=False, needs_layout_passes=False
    ),
    scratch_types=[
        pltpu.VMEM((PER_W,), jnp.int32),
        pltpu.VMEM((2, GSIZE), jnp.int32),
        pltpu.VMEM((2, GSIZE, EMBED), jnp.float32),
        pltpu.VMEM((2, EMBED, BLK), jnp.float32),
        pltpu.SemaphoreType.DMA,
        pltpu.SemaphoreType.DMA,
        pltpu.SemaphoreType.DMA,
    ],
)
def _embed_sc(idx_hbm, tab_hbm, out_hbm, idx_all, lst_v, g_v, gt_v,
              sem_g, ss0, ss1):
    u = lax.axis_index("s") * NC + lax.axis_index("c")

    pltpu.sync_copy(idx_hbm.at[pl.ds(u * PER_W, PER_W)], idx_all)

    lane = lax.iota(jnp.int32, 16) * T
    iota16 = lax.iota(jnp.int32, 16)

    def build_and_fire(g, m):
        t0 = g * SUB

        def zbuild(z, carry):
            for k in range(BLK // 16):
                pos = lane + (k * 16 * T + t0 + z)
                lst_v[m, pl.dslice(z * BLK + k * 16, 16)] = plsc.load_gather(
                    idx_all, [pos]
                )
            return carry

        lax.fori_loop(0, SUB, zbuild, 0)
        pltpu.async_copy(tab_hbm.at[lst_v.at[m]], g_v.at[m], sem_g)

    def wait_gather():
        pltpu.make_async_copy(
            tab_hbm.at[pl.ds(0, GSIZE)], g_v.at[0], sem_g
        ).wait()

    def transpose(m, z, d):
        def jbody(j, carry):
            cj = jnp.full((16,), j, jnp.int32)
            for k in range(BLK // 16):
                rows = iota16 + (z * BLK + k * 16)
                gt_v[d, j, pl.dslice(k * 16, 16)] = plsc.load_gather(
                    g_v.at[m], [rows, cj]
                )
            return carry

        lax.fori_loop(0, EMBED, jbody, 0)

    def fire_stores(t, d, sem):
        for s in range(EMBED // 8):
            pltpu.async_copy(
                gt_v.at[d, pl.ds(8 * s, 8)], out_hbm.at[t, s, u], sem
            )

    def wait_stores(sem):
        for _ in range(EMBED // 8):
            pltpu.make_async_copy(
                gt_v.at[0, pl.ds(0, 8)], out_hbm.at[0, 0, 0], sem
            ).wait()

    build_and_fire(0, 0)
    build_and_fire(1, 1)

    def gbody(g, carry):
        m = lax.rem(g, 2)
        wait_gather()

        def zbody(z, carry2):
            t = g * SUB + z
            d = lax.rem(z, 2)

            @pl.when(jnp.logical_and(d == 0, t >= 2))
            def _():
                wait_stores(ss0)

            @pl.when(jnp.logical_and(d == 1, t >= 2))
            def _():
                wait_stores(ss1)

            transpose(m, z, d)

            @pl.when(d == 0)
            def _():
                fire_stores(t, d, ss0)

            @pl.when(d == 1)
            def _():
                fire_stores(t, d, ss1)

            return carry2

        lax.fori_loop(0, SUB, zbody, 0)

        @pl.when(g + 2 < NGROUP)
        def _():
            build_and_fire(g + 2, m)

        return carry

    lax.fori_loop(0, NGROUP, gbody, 0)
    wait_stores(ss0)
    wait_stores(ss1)


def kernel(tokens, W_E):
    idx = tokens.reshape(N).astype(jnp.int32)
    out5 = _embed_sc(idx, W_E)
    return out5.transpose(2, 4, 0, 1, 3).reshape(B, T, EMBED)


# speedup vs baseline: 1.0040x; 1.0040x over previous
import functools

import jax
import jax.numpy as jnp
from jax import lax
from jax.experimental import pallas as pl
from jax.experimental.pallas import tpu as pltpu
from jax.experimental.pallas import tpu_sc as plsc

VOCAB = 1000000
EMBED = 32
B, T = 4096, 200
N = B * T

_info = plsc.get_sparse_core_info()
NC, NS = _info.num_cores, _info.num_subcores
NW = NC * NS
BLK = 128
SUB = 10
GSIZE = BLK * SUB
PER_W = N // NW
NGROUP = PER_W // GSIZE

_mesh = plsc.VectorSubcoreMesh(core_axis_name="c", subcore_axis_name="s")


@functools.partial(
    pl.kernel,
    mesh=_mesh,
    out_type=jax.ShapeDtypeStruct((T, EMBED // 8, B // BLK, 8, BLK), jnp.float32),
    compiler_params=pltpu.CompilerParams(
        use_tc_tiling_on_sc=False, needs_layout_passes=False
    ),
    scratch_types=[
        pltpu.VMEM((PER_W,), jnp.int32),
        pltpu.VMEM((2, GSIZE), jnp.int32),
        pltpu.VMEM((2, GSIZE, EMBED), jnp.float32),
        pltpu.VMEM((2, EMBED, BLK), jnp.float32),
        pltpu.SemaphoreType.DMA,
        pltpu.SemaphoreType.DMA,
        pltpu.SemaphoreType.DMA,
    ],
)
def _embed_sc(idx_hbm, tab_hbm, out_hbm, idx_all, lst_v, g_v, gt_v,
              sem_g, ss0, ss1):
    u = lax.axis_index("s") * NC + lax.axis_index("c")

    pltpu.sync_copy(idx_hbm.at[pl.ds(u * PER_W, PER_W)], idx_all)

    lane = lax.iota(jnp.int32, 16) * T
    iota16 = lax.iota(jnp.int32, 16)

    def build_and_fire(g, m):
        t0 = g * SUB

        def zbuild(z, carry):
            for k in range(BLK // 16):
                pos = lane + (k * 16 * T + t0 + z)
                lst_v[m, pl.dslice(z * BLK + k * 16, 16)] = plsc.load_gather(
                    idx_all, [pos]
                )
            return carry

        lax.fori_loop(0, SUB, zbuild, 0)
        pltpu.async_copy(tab_hbm.at[lst_v.at[m]], g_v.at[m], sem_g)

    def wait_gather():
        pltpu.make_async_copy(
            tab_hbm.at[pl.ds(0, GSIZE)], g_v.at[0], sem_g
        ).wait()

    def transpose(m, z, d):
        def jbody(j, carry):
            cj = jnp.full((16,), j, jnp.int32)
            for k in range(BLK // 16):
                rows = iota16 + (z * BLK + k * 16)
                gt_v[d, j, pl.dslice(k * 16, 16)] = plsc.load_gather(
                    g_v.at[m], [rows, cj]
                )
            return carry

        lax.fori_loop(0, EMBED, jbody, 0, unroll=8)

    def fire_stores(t, d, sem):
        for s in range(EMBED // 8):
            pltpu.async_copy(
                gt_v.at[d, pl.ds(8 * s, 8)], out_hbm.at[t, s, u], sem
            )

    def wait_stores(sem):
        for _ in range(EMBED // 8):
            pltpu.make_async_copy(
                gt_v.at[0, pl.ds(0, 8)], out_hbm.at[0, 0, 0], sem
            ).wait()

    build_and_fire(0, 0)
    build_and_fire(1, 1)

    def gbody(g, carry):
        m = lax.rem(g, 2)
        wait_gather()

        def zbody(z, carry2):
            t = g * SUB + z
            d = lax.rem(z, 2)

            @pl.when(jnp.logical_and(d == 0, t >= 2))
            def _():
                wait_stores(ss0)

            @pl.when(jnp.logical_and(d == 1, t >= 2))
            def _():
                wait_stores(ss1)

            transpose(m, z, d)

            @pl.when(d == 0)
            def _():
                fire_stores(t, d, ss0)

            @pl.when(d == 1)
            def _():
                fire_stores(t, d, ss1)

            return carry2

        lax.fori_loop(0, SUB, zbody, 0)

        @pl.when(g + 2 < NGROUP)
        def _():
            build_and_fire(g + 2, m)

        return carry

    lax.fori_loop(0, NGROUP, gbody, 0)
    wait_stores(ss0)
    wait_stores(ss1)


def kernel(tokens, W_E):
    idx = tokens.reshape(N).astype(jnp.int32)
    out5 = _embed_sc(idx, W_E)
    return out5.transpose(2, 4, 0, 1, 3).reshape(B, T, EMBED)
